# baseline (speedup 1.0000x reference)
import functools

import jax
import jax.numpy as jnp
from jax import lax
from jax.experimental import pallas as pl
from jax.experimental.pallas import tpu as pltpu
from jax.experimental.pallas import tpu_sc as plsc

B, C, N, O = 8, 4, 2048, 64
KMAX = 40
RCOLS = 128
T_C = 256
NC, NS = 2, 16
NW = NC * NS
NGRP = (B * N) // 16
GPT = NGRP // NW
G1, G2 = 128, 8


def _pairwise_kernel(xr_ref, x_ref, xc_ref, p_ref, gm_ref, sg_ref):
    xr = xr_ref[0]
    xa = x_ref[0]
    xc = xc_ref[0]
    m = lax.dot_general(xr, xa, (((1,), (0,)), ((), ())),
                        preferred_element_type=jnp.float32)
    xsq_r = jnp.sum(xr * xr, axis=1, keepdims=True)
    xsq_a = jnp.sum(xa * xa, axis=0, keepdims=True)
    p_ref[0, 0] = (2.0 * m - xsq_r) - xsq_a
    mo = lax.dot_general(xa, xc, (((0,), (0,)), ((), ())),
                         preferred_element_type=jnp.float32)
    po = (2.0 * mo - jnp.transpose(xsq_a)) - jnp.transpose(xsq_r)
    gm = jnp.max(po.reshape(G1, 16, RCOLS), axis=1)
    gm_ref[0, 0] = jnp.transpose(gm)
    sg = jnp.max(gm.reshape(G2, 16, RCOLS), axis=1)
    sg_ref[0, 0] = jnp.transpose(sg)


def _pairwise(xt, x):
    return pl.pallas_call(
        _pairwise_kernel,
        grid=(B, N // RCOLS),
        in_specs=[
            pl.BlockSpec((1, RCOLS, C), lambda b, i: (b, i, 0)),
            pl.BlockSpec((1, C, N), lambda b, i: (b, 0, 0)),
            pl.BlockSpec((1, C, RCOLS), lambda b, i: (b, 0, i)),
        ],
        out_specs=[
            pl.BlockSpec((1, 1, RCOLS, N), lambda b, i: (b, i, 0, 0)),
            pl.BlockSpec((1, 1, RCOLS, G1), lambda b, i: (b, i, 0, 0)),
            pl.BlockSpec((1, 1, RCOLS, G2), lambda b, i: (b, i, 0, 0)),
        ],
        out_shape=[
            jax.ShapeDtypeStruct((B, N // RCOLS, RCOLS, N), jnp.float32),
            jax.ShapeDtypeStruct((B, N // RCOLS, RCOLS, G1), jnp.float32),
            jax.ShapeDtypeStruct((B, N // RCOLS, RCOLS, G2), jnp.float32),
        ],
    )(xt, x, x)


_mesh = plsc.VectorSubcoreMesh(core_axis_name="c", subcore_axis_name="s")


@functools.partial(
    pl.kernel,
    mesh=_mesh,
    compiler_params=pltpu.CompilerParams(use_tc_tiling_on_sc=False,
                                         needs_layout_passes=False),
    out_type=jax.ShapeDtypeStruct((B, C, KMAX, N), jnp.float32),
    scratch_types=[
        pltpu.VMEM((16, N), jnp.float32),
        pltpu.VMEM((B, C, N), jnp.float32),
        pltpu.VMEM((16, G1), jnp.float32),
        pltpu.VMEM((16, G2), jnp.float32),
        pltpu.VMEM((C, KMAX, 16), jnp.float32),
    ],
)
def _topk_gather(pt_hbm, gm_hbm, sg_hbm, x_hbm, xg_hbm, ptv, xv, gmv, sgv, xgv):
    wid = lax.axis_index("c") * NS + lax.axis_index("s")
    lane = lax.iota(jnp.int32, 16)
    neg = jnp.full((16,), -jnp.inf, jnp.float32)

    pltpu.sync_copy(x_hbm, xv)

    def group_body(g, carry):
        grp = wid * GPT + g
        b = grp // (N // 16)
        rem = grp % (N // 16)
        gi = rem // 8
        r0 = (rem % 8) * 16
        i0 = rem * 16
        pltpu.sync_copy(pt_hbm.at[b, gi, pl.ds(r0, 16), :], ptv)
        pltpu.sync_copy(gm_hbm.at[b, gi, pl.ds(r0, 16), :], gmv)
        pltpu.sync_copy(sg_hbm.at[b, gi, pl.ds(r0, 16), :], sgv)

        bt = jnp.full((16,), b, jnp.int32)

        def ex_body(t, c1):
            m0 = plsc.load_gather(sgv, [lane, jnp.zeros((16,), jnp.int32)])
            sidx = jnp.zeros((16,), jnp.int32)
            for s in range(1, G2):
                sv = jnp.full((16,), s, jnp.int32)
                v = plsc.load_gather(sgv, [lane, sv])
                cm = v > m0
                m0 = jnp.where(cm, v, m0)
                sidx = jnp.where(cm, sv, sidx)
            gbase = sidx * 16
            gvals = []
            m1 = plsc.load_gather(gmv, [lane, gbase])
            gidx = gbase
            gvals.append((gbase, m1))
            for t2 in range(1, 16):
                gv = gbase + t2
                v = plsc.load_gather(gmv, [lane, gv])
                gvals.append((gv, v))
                cm = v > m1
                m1 = jnp.where(cm, v, m1)
                gidx = jnp.where(cm, gv, gidx)
            jbase = gidx * 16
            evals = []
            m2 = plsc.load_gather(ptv, [lane, jbase])
            jidx = jbase
            evals.append((jbase, m2))
            for t3 in range(1, 16):
                jv = jbase + t3
                v = plsc.load_gather(ptv, [lane, jv])
                evals.append((jv, v))
                cm = v > m2
                m2 = jnp.where(cm, v, m2)
                jidx = jnp.where(cm, jv, jidx)
            tt = jnp.full((16,), t, jnp.int32)
            for cc in range(C):
                ccv = jnp.full((16,), cc, jnp.int32)
                xval = plsc.load_gather(xv, [bt, ccv, jidx])
                plsc.store_scatter(xgv, [ccv, tt, lane], xval)
            plsc.store_scatter(ptv, [lane, jidx], neg)
            ngm = neg
            for (jv, v) in evals:
                ngm = jnp.maximum(ngm, jnp.where(jv == jidx, neg, v))
            plsc.store_scatter(gmv, [lane, gidx], ngm)
            nsg = neg
            for (gv, v) in gvals:
                nsg = jnp.maximum(nsg, jnp.where(gv == gidx, ngm, v))
            plsc.store_scatter(sgv, [lane, sidx], nsg)
            return c1
        lax.fori_loop(0, KMAX, ex_body, 0)

        pltpu.sync_copy(xgv, xg_hbm.at[b, :, :, pl.ds(i0, 16)])
        return carry
    lax.fori_loop(0, GPT, group_body, 0)


def _leaky(h):
    return jnp.where(h >= 0, h, 0.2 * h)


def _scales_kernel(xg_ref, x_ref, acat_ref, dcat_ref, bcat_ref,
                   wa1_ref, ba1_ref, wa2_ref, ba2_ref,
                   fused_ref, w_ref):
    x = x_ref[0]
    acat = acat_ref[...]
    zcat = jnp.dot(dcat_ref[...], x, preferred_element_type=jnp.float32) \
        + bcat_ref[...]

    feats = []
    m = None
    for kk in range(KMAX):
        y = jnp.dot(acat, xg_ref[0, :, kk, :],
                    preferred_element_type=jnp.float32)
        m = y if m is None else jnp.maximum(m, y)
        if kk == 9:
            feats.append(_leaky(m[0:O] + zcat[0:O]))
        elif kk == 19:
            feats.append(_leaky(m[O:2 * O] + zcat[O:2 * O]))
        elif kk == 39:
            feats.append(_leaky(m[2 * O:3 * O] + zcat[2 * O:3 * O]))

    cc = jnp.concatenate(feats, axis=0)
    a = jnp.dot(wa1_ref[...], cc, preferred_element_type=jnp.float32) + ba1_ref[...]
    a = jnp.maximum(a, 0.0)
    logits = jnp.dot(wa2_ref[...], a, preferred_element_type=jnp.float32) + ba2_ref[...]
    lmax = jnp.max(logits, axis=0, keepdims=True)
    ex = jnp.exp(logits - lmax)
    w = ex / jnp.sum(ex, axis=0, keepdims=True)
    w_ref[0] = w
    fused = w[0:1] * feats[0] + w[1:2] * feats[1] + w[2:3] * feats[2]
    fused_ref[0] = fused


def _scales(xg, x, acat, dcat, bcat, wa1p, ba1p, wa2, ba2):
    return pl.pallas_call(
        _scales_kernel,
        grid=(B, N // T_C),
        in_specs=[
            pl.BlockSpec((1, C, KMAX, T_C), lambda b, t: (b, 0, 0, t)),
            pl.BlockSpec((1, C, T_C), lambda b, t: (b, 0, t)),
            pl.BlockSpec((3 * O, C), lambda b, t: (0, 0)),
            pl.BlockSpec((3 * O, C), lambda b, t: (0, 0)),
            pl.BlockSpec((3 * O, 1), lambda b, t: (0, 0)),
            pl.BlockSpec((O, 3 * O), lambda b, t: (0, 0)),
            pl.BlockSpec((O, 1), lambda b, t: (0, 0)),
            pl.BlockSpec((3, O), lambda b, t: (0, 0)),
            pl.BlockSpec((3, 1), lambda b, t: (0, 0)),
        ],
        out_specs=[
            pl.BlockSpec((1, O, T_C), lambda b, t: (b, 0, t)),
            pl.BlockSpec((1, 3, T_C), lambda b, t: (b, 0, t)),
        ],
        out_shape=[
            jax.ShapeDtypeStruct((B, O, N), jnp.float32),
            jax.ShapeDtypeStruct((B, 3, N), jnp.float32),
        ],
    )(xg, x, acat, dcat, bcat, wa1p, ba1p, wa2, ba2)


def kernel(x, W0, g0, b0, W1, g1, b1, W2, g2, b2, Wa1, ba1, ga, bba, Wa2, ba2):
    acat = jnp.concatenate([g0[:, None] * W0[:, :C],
                            g1[:, None] * W1[:, :C],
                            g2[:, None] * W2[:, :C]], axis=0)
    dcat = jnp.concatenate([g0[:, None] * (W0[:, C:] - W0[:, :C]),
                            g1[:, None] * (W1[:, C:] - W1[:, :C]),
                            g2[:, None] * (W2[:, C:] - W2[:, :C])], axis=0)
    bcat = jnp.concatenate([b0, b1, b2], axis=0)[:, None]
    wa1p = ga[:, None] * Wa1
    ba1p = (ga * ba1 + bba)[:, None]
    ba2c = ba2[:, None]

    xt = jnp.transpose(x, (0, 2, 1))
    pt, gm, sg = _pairwise(xt, x)
    xg = _topk_gather(pt, gm, sg, x)
    fused, w = _scales(xg, x, acat, dcat, bcat, wa1p, ba1p, Wa2, ba2c)
    return fused, w


# speedup vs baseline: 2.1097x; 2.1097x over previous
import functools

import jax
import jax.numpy as jnp
from jax import lax
from jax.experimental import pallas as pl
from jax.experimental.pallas import tpu as pltpu
from jax.experimental.pallas import tpu_sc as plsc

B, C, N, O = 8, 4, 2048, 64
KMAX = 40
RCOLS = 128
T_C = 256
NC, NS = 2, 16
NW = NC * NS
NGRP = (B * N) // 16
GPT = NGRP // NW
G1, G2 = 128, 8


def _pairwise_kernel(xt_ref, xc_ref, p_ref, gm_ref, sg_ref):
    xt = xt_ref[0]
    xc = xc_ref[0]
    m = lax.dot_general(xt, xc, (((1,), (0,)), ((), ())),
                        preferred_element_type=jnp.float32)
    xsq_t = jnp.sum(xt * xt, axis=1, keepdims=True)
    xsq_r = jnp.sum(xc * xc, axis=0, keepdims=True)
    p = (2.0 * m - xsq_t) - xsq_r
    p_ref[0, 0] = p
    gm = jnp.max(p.reshape(G1, 16, RCOLS), axis=1)
    gm_ref[0, 0] = gm
    sg_ref[0, 0] = jnp.max(gm.reshape(G2, 16, RCOLS), axis=1)


def _pairwise(xt, x):
    return pl.pallas_call(
        _pairwise_kernel,
        grid=(B, N // RCOLS),
        in_specs=[
            pl.BlockSpec((1, N, C), lambda b, i: (b, 0, 0)),
            pl.BlockSpec((1, C, RCOLS), lambda b, i: (b, 0, i)),
        ],
        out_specs=[
            pl.BlockSpec((1, 1, N, RCOLS), lambda b, i: (b, i, 0, 0)),
            pl.BlockSpec((1, 1, G1, RCOLS), lambda b, i: (b, i, 0, 0)),
            pl.BlockSpec((1, 1, G2, RCOLS), lambda b, i: (b, i, 0, 0)),
        ],
        out_shape=[
            jax.ShapeDtypeStruct((B, N // RCOLS, N, RCOLS), jnp.float32),
            jax.ShapeDtypeStruct((B, N // RCOLS, G1, RCOLS), jnp.float32),
            jax.ShapeDtypeStruct((B, N // RCOLS, G2, RCOLS), jnp.float32),
        ],
    )(xt, x)


_mesh = plsc.VectorSubcoreMesh(core_axis_name="c", subcore_axis_name="s")


@functools.partial(
    pl.kernel,
    mesh=_mesh,
    compiler_params=pltpu.CompilerParams(use_tc_tiling_on_sc=False,
                                         needs_layout_passes=False),
    out_type=jax.ShapeDtypeStruct((B, C, KMAX, N), jnp.float32),
    scratch_types=[
        pltpu.VMEM((N, 16), jnp.float32),
        pltpu.VMEM((N, 16), jnp.float32),
        pltpu.VMEM((C, N), jnp.float32),
        pltpu.VMEM((G1, 16), jnp.float32),
        pltpu.VMEM((G1, 16), jnp.float32),
        pltpu.VMEM((G2, 16), jnp.float32),
        pltpu.VMEM((G2, 16), jnp.float32),
        pltpu.VMEM((C, KMAX, 16), jnp.float32),
        pltpu.SemaphoreType.DMA,
        pltpu.SemaphoreType.DMA,
    ],
)
def _topk_gather(pt_hbm, gm_hbm, sg_hbm, x_hbm, xg_hbm,
                 ptv0, ptv1, xv, gmv0, gmv1, sgv0, sgv1, xgv, sem0, sem1):
    wid = lax.axis_index("c") * NS + lax.axis_index("s")
    lane = lax.iota(jnp.int32, 16)
    neg = jnp.full((16,), -jnp.inf, jnp.float32)

    b_tec = wid // ((N // 16) // GPT)
    pltpu.sync_copy(x_hbm.at[b_tec], xv)

    def addr(g):
        grp = wid * GPT + jnp.minimum(g, GPT - 1)
        b = grp // (N // 16)
        rem = grp % (N // 16)
        return b, rem // 8, (rem % 8) * 16

    def start_fetch(g, ptv, gmv, sgv, sem):
        b, gi, r0 = addr(g)
        pltpu.async_copy(pt_hbm.at[b, gi, :, pl.ds(r0, 16)], ptv, sem)
        pltpu.async_copy(gm_hbm.at[b, gi, :, pl.ds(r0, 16)], gmv, sem)
        pltpu.async_copy(sg_hbm.at[b, gi, :, pl.ds(r0, 16)], sgv, sem)

    def drain(ptv, gmv, sgv, sem):
        pltpu.make_async_copy(pt_hbm.at[0, 0, :, pl.ds(0, 16)], ptv, sem).wait()
        pltpu.make_async_copy(gm_hbm.at[0, 0, :, pl.ds(0, 16)], gmv, sem).wait()
        pltpu.make_async_copy(sg_hbm.at[0, 0, :, pl.ds(0, 16)], sgv, sem).wait()

    def process(g, ptv, gmv, sgv):
        grp = wid * GPT + g
        b = grp // (N // 16)
        i0 = (grp % (N // 16)) * 16

        def ex_body(t, c1):
            m0 = plsc.load_gather(sgv, [jnp.zeros((16,), jnp.int32), lane])
            sidx = jnp.zeros((16,), jnp.int32)
            for s in range(1, G2):
                sv = jnp.full((16,), s, jnp.int32)
                v = plsc.load_gather(sgv, [sv, lane])
                cm = v > m0
                m0 = jnp.where(cm, v, m0)
                sidx = jnp.where(cm, sv, sidx)
            gbase = sidx * 16
            gvals = []
            m1 = plsc.load_gather(gmv, [gbase, lane])
            gidx = gbase
            gvals.append((gbase, m1))
            for t2 in range(1, 16):
                gv = gbase + t2
                v = plsc.load_gather(gmv, [gv, lane])
                gvals.append((gv, v))
                cm = v > m1
                m1 = jnp.where(cm, v, m1)
                gidx = jnp.where(cm, gv, gidx)
            jbase = gidx * 16
            evals = []
            m2 = plsc.load_gather(ptv, [jbase, lane])
            jidx = jbase
            evals.append((jbase, m2))
            for t3 in range(1, 16):
                jv = jbase + t3
                v = plsc.load_gather(ptv, [jv, lane])
                evals.append((jv, v))
                cm = v > m2
                m2 = jnp.where(cm, v, m2)
                jidx = jnp.where(cm, jv, jidx)
            tt = jnp.full((16,), t, jnp.int32)
            for cc in range(C):
                ccv = jnp.full((16,), cc, jnp.int32)
                xval = plsc.load_gather(xv, [ccv, jidx])
                plsc.store_scatter(xgv, [ccv, tt, lane], xval)
            plsc.store_scatter(ptv, [jidx, lane], neg)
            ngm = neg
            for (jv, v) in evals:
                ngm = jnp.maximum(ngm, jnp.where(jv == jidx, neg, v))
            plsc.store_scatter(gmv, [gidx, lane], ngm)
            nsg = neg
            for (gv, v) in gvals:
                nsg = jnp.maximum(nsg, jnp.where(gv == gidx, ngm, v))
            plsc.store_scatter(sgv, [sidx, lane], nsg)
            return c1
        lax.fori_loop(0, KMAX, ex_body, 0)

        pltpu.sync_copy(xgv, xg_hbm.at[b, :, :, pl.ds(i0, 16)])

    start_fetch(0, ptv0, gmv0, sgv0, sem0)

    def pair_body(h, carry):
        g = h * 2
        start_fetch(g + 1, ptv1, gmv1, sgv1, sem1)
        drain(ptv0, gmv0, sgv0, sem0)
        process(g, ptv0, gmv0, sgv0)
        start_fetch(g + 2, ptv0, gmv0, sgv0, sem0)
        drain(ptv1, gmv1, sgv1, sem1)
        process(g + 1, ptv1, gmv1, sgv1)
        return carry
    lax.fori_loop(0, GPT // 2, pair_body, 0)

    drain(ptv0, gmv0, sgv0, sem0)


def _leaky(h):
    return jnp.where(h >= 0, h, 0.2 * h)


def _scales_kernel(xg_ref, x_ref, acat_ref, dcat_ref, bcat_ref,
                   wa1_ref, ba1_ref, wa2_ref, ba2_ref,
                   fused_ref, w_ref):
    x = x_ref[0]
    acat = acat_ref[...]
    zcat = jnp.dot(dcat_ref[...], x, preferred_element_type=jnp.float32) \
        + bcat_ref[...]

    feats = []
    m = None
    for kk in range(KMAX):
        y = jnp.dot(acat, xg_ref[0, :, kk, :],
                    preferred_element_type=jnp.float32)
        m = y if m is None else jnp.maximum(m, y)
        if kk == 9:
            feats.append(_leaky(m[0:O] + zcat[0:O]))
        elif kk == 19:
            feats.append(_leaky(m[O:2 * O] + zcat[O:2 * O]))
        elif kk == 39:
            feats.append(_leaky(m[2 * O:3 * O] + zcat[2 * O:3 * O]))

    cc = jnp.concatenate(feats, axis=0)
    a = jnp.dot(wa1_ref[...], cc, preferred_element_type=jnp.float32) + ba1_ref[...]
    a = jnp.maximum(a, 0.0)
    logits = jnp.dot(wa2_ref[...], a, preferred_element_type=jnp.float32) + ba2_ref[...]
    lmax = jnp.max(logits, axis=0, keepdims=True)
    ex = jnp.exp(logits - lmax)
    w = ex / jnp.sum(ex, axis=0, keepdims=True)
    w_ref[0] = w
    fused = w[0:1] * feats[0] + w[1:2] * feats[1] + w[2:3] * feats[2]
    fused_ref[0] = fused


def _scales(xg, x, acat, dcat, bcat, wa1p, ba1p, wa2, ba2):
    return pl.pallas_call(
        _scales_kernel,
        grid=(B, N // T_C),
        in_specs=[
            pl.BlockSpec((1, C, KMAX, T_C), lambda b, t: (b, 0, 0, t)),
            pl.BlockSpec((1, C, T_C), lambda b, t: (b, 0, t)),
            pl.BlockSpec((3 * O, C), lambda b, t: (0, 0)),
            pl.BlockSpec((3 * O, C), lambda b, t: (0, 0)),
            pl.BlockSpec((3 * O, 1), lambda b, t: (0, 0)),
            pl.BlockSpec((O, 3 * O), lambda b, t: (0, 0)),
            pl.BlockSpec((O, 1), lambda b, t: (0, 0)),
            pl.BlockSpec((3, O), lambda b, t: (0, 0)),
            pl.BlockSpec((3, 1), lambda b, t: (0, 0)),
        ],
        out_specs=[
            pl.BlockSpec((1, O, T_C), lambda b, t: (b, 0, t)),
            pl.BlockSpec((1, 3, T_C), lambda b, t: (b, 0, t)),
        ],
        out_shape=[
            jax.ShapeDtypeStruct((B, O, N), jnp.float32),
            jax.ShapeDtypeStruct((B, 3, N), jnp.float32),
        ],
    )(xg, x, acat, dcat, bcat, wa1p, ba1p, wa2, ba2)


def kernel(x, W0, g0, b0, W1, g1, b1, W2, g2, b2, Wa1, ba1, ga, bba, Wa2, ba2):
    acat = jnp.concatenate([g0[:, None] * W0[:, :C],
                            g1[:, None] * W1[:, :C],
                            g2[:, None] * W2[:, :C]], axis=0)
    dcat = jnp.concatenate([g0[:, None] * (W0[:, C:] - W0[:, :C]),
                            g1[:, None] * (W1[:, C:] - W1[:, :C]),
                            g2[:, None] * (W2[:, C:] - W2[:, :C])], axis=0)
    bcat = jnp.concatenate([b0, b1, b2], axis=0)[:, None]
    wa1p = ga[:, None] * Wa1
    ba1p = (ga * ba1 + bba)[:, None]
    ba2c = ba2[:, None]

    xt = jnp.transpose(x, (0, 2, 1))
    pt, gm, sg = _pairwise(xt, x)
    xg = _topk_gather(pt, gm, sg, x)
    fused, w = _scales(xg, x, acat, dcat, bcat, wa1p, ba1p, Wa2, ba2c)
    return fused, w


# speedup vs baseline: 2.2749x; 1.0783x over previous
import functools

import jax
import jax.numpy as jnp
from jax import lax
from jax.experimental import pallas as pl
from jax.experimental.pallas import tpu as pltpu
from jax.experimental.pallas import tpu_sc as plsc

B, C, N, O = 8, 4, 2048, 64
KMAX = 40
RCOLS = 128
T_C = 256
NC, NS = 2, 16
NW = NC * NS
NGRP = (B * N) // 16
GPT = NGRP // NW
G1, G2 = 128, 8


def _pairwise_kernel(xt_ref, xc_ref, p_ref, gm_ref, sg_ref):
    xt = xt_ref[0]
    xc = xc_ref[0]
    m = lax.dot_general(xt, xc, (((1,), (0,)), ((), ())),
                        preferred_element_type=jnp.float32)
    xsq_t = jnp.sum(xt * xt, axis=1, keepdims=True)
    xsq_r = jnp.sum(xc * xc, axis=0, keepdims=True)
    p = (2.0 * m - xsq_t) - xsq_r
    p_ref[0, 0] = p
    gm = jnp.max(p.reshape(G1, 16, RCOLS), axis=1)
    gm_ref[0, 0] = gm
    sg_ref[0, 0] = jnp.max(gm.reshape(G2, 16, RCOLS), axis=1)


def _pairwise(xt, x, hb):
    return pl.pallas_call(
        _pairwise_kernel,
        grid=(hb, N // RCOLS),
        in_specs=[
            pl.BlockSpec((1, N, C), lambda b, i: (b, 0, 0)),
            pl.BlockSpec((1, C, RCOLS), lambda b, i: (b, 0, i)),
        ],
        out_specs=[
            pl.BlockSpec((1, 1, N, RCOLS), lambda b, i: (b, i, 0, 0)),
            pl.BlockSpec((1, 1, G1, RCOLS), lambda b, i: (b, i, 0, 0)),
            pl.BlockSpec((1, 1, G2, RCOLS), lambda b, i: (b, i, 0, 0)),
        ],
        out_shape=[
            jax.ShapeDtypeStruct((hb, N // RCOLS, N, RCOLS), jnp.float32),
            jax.ShapeDtypeStruct((hb, N // RCOLS, G1, RCOLS), jnp.float32),
            jax.ShapeDtypeStruct((hb, N // RCOLS, G2, RCOLS), jnp.float32),
        ],
    )(xt, x)


_mesh = plsc.VectorSubcoreMesh(core_axis_name="c", subcore_axis_name="s")


def _make_topk(hb):
  gpt = (hb * N // 16) // NW

  @functools.partial(
      pl.kernel,
      mesh=_mesh,
      compiler_params=pltpu.CompilerParams(use_tc_tiling_on_sc=False,
                                           needs_layout_passes=False),
      out_type=jax.ShapeDtypeStruct((hb, C, KMAX, N), jnp.float32),
      scratch_types=[
          pltpu.VMEM((N, 16), jnp.float32),
          pltpu.VMEM((N, 16), jnp.float32),
          pltpu.VMEM((C, N), jnp.float32),
          pltpu.VMEM((G1, 16), jnp.float32),
          pltpu.VMEM((G1, 16), jnp.float32),
          pltpu.VMEM((G2, 16), jnp.float32),
          pltpu.VMEM((G2, 16), jnp.float32),
          pltpu.VMEM((C, KMAX, 16), jnp.float32),
          pltpu.SemaphoreType.DMA,
          pltpu.SemaphoreType.DMA,
      ],
  )
  def _topk_gather(pt_hbm, gm_hbm, sg_hbm, x_hbm, xg_hbm,
                   ptv0, ptv1, xv, gmv0, gmv1, sgv0, sgv1, xgv, sem0, sem1):
    wid = lax.axis_index("c") * NS + lax.axis_index("s")
    lane = lax.iota(jnp.int32, 16)
    neg = jnp.full((16,), -jnp.inf, jnp.float32)

    b_tec = wid // ((N // 16) // gpt)
    pltpu.sync_copy(x_hbm.at[b_tec], xv)

    def addr(g):
        grp = wid * gpt + jnp.minimum(g, gpt - 1)
        b = grp // (N // 16)
        rem = grp % (N // 16)
        return b, rem // 8, (rem % 8) * 16

    def start_fetch(g, ptv, gmv, sgv, sem):
        b, gi, r0 = addr(g)
        pltpu.async_copy(pt_hbm.at[b, gi, :, pl.ds(r0, 16)], ptv, sem)
        pltpu.async_copy(gm_hbm.at[b, gi, :, pl.ds(r0, 16)], gmv, sem)
        pltpu.async_copy(sg_hbm.at[b, gi, :, pl.ds(r0, 16)], sgv, sem)

    def drain(ptv, gmv, sgv, sem):
        pltpu.make_async_copy(pt_hbm.at[0, 0, :, pl.ds(0, 16)], ptv, sem).wait()
        pltpu.make_async_copy(gm_hbm.at[0, 0, :, pl.ds(0, 16)], gmv, sem).wait()
        pltpu.make_async_copy(sg_hbm.at[0, 0, :, pl.ds(0, 16)], sgv, sem).wait()

    def process(g, ptv, gmv, sgv):
        grp = wid * gpt + g
        b = grp // (N // 16)
        i0 = (grp % (N // 16)) * 16

        def ex_body(t, c1):
            m0 = plsc.load_gather(sgv, [jnp.zeros((16,), jnp.int32), lane])
            sidx = jnp.zeros((16,), jnp.int32)
            for s in range(1, G2):
                sv = jnp.full((16,), s, jnp.int32)
                v = plsc.load_gather(sgv, [sv, lane])
                cm = v > m0
                m0 = jnp.where(cm, v, m0)
                sidx = jnp.where(cm, sv, sidx)
            gbase = sidx * 16
            gvals = []
            m1 = plsc.load_gather(gmv, [gbase, lane])
            gidx = gbase
            gvals.append((gbase, m1))
            for t2 in range(1, 16):
                gv = gbase + t2
                v = plsc.load_gather(gmv, [gv, lane])
                gvals.append((gv, v))
                cm = v > m1
                m1 = jnp.where(cm, v, m1)
                gidx = jnp.where(cm, gv, gidx)
            jbase = gidx * 16
            evals = []
            m2 = plsc.load_gather(ptv, [jbase, lane])
            jidx = jbase
            evals.append((jbase, m2))
            for t3 in range(1, 16):
                jv = jbase + t3
                v = plsc.load_gather(ptv, [jv, lane])
                evals.append((jv, v))
                cm = v > m2
                m2 = jnp.where(cm, v, m2)
                jidx = jnp.where(cm, jv, jidx)
            tt = jnp.full((16,), t, jnp.int32)
            for cc in range(C):
                ccv = jnp.full((16,), cc, jnp.int32)
                xval = plsc.load_gather(xv, [ccv, jidx])
                plsc.store_scatter(xgv, [ccv, tt, lane], xval)
            plsc.store_scatter(ptv, [jidx, lane], neg)
            ngm = neg
            for (jv, v) in evals:
                ngm = jnp.maximum(ngm, jnp.where(jv == jidx, neg, v))
            plsc.store_scatter(gmv, [gidx, lane], ngm)
            nsg = neg
            for (gv, v) in gvals:
                nsg = jnp.maximum(nsg, jnp.where(gv == gidx, ngm, v))
            plsc.store_scatter(sgv, [sidx, lane], nsg)
            return c1
        lax.fori_loop(0, KMAX, ex_body, 0)

        pltpu.sync_copy(xgv, xg_hbm.at[b, :, :, pl.ds(i0, 16)])

    start_fetch(0, ptv0, gmv0, sgv0, sem0)

    def pair_body(h, carry):
        g = h * 2
        start_fetch(g + 1, ptv1, gmv1, sgv1, sem1)
        drain(ptv0, gmv0, sgv0, sem0)
        process(g, ptv0, gmv0, sgv0)
        start_fetch(g + 2, ptv0, gmv0, sgv0, sem0)
        drain(ptv1, gmv1, sgv1, sem1)
        process(g + 1, ptv1, gmv1, sgv1)
        return carry
    lax.fori_loop(0, gpt // 2, pair_body, 0)

    drain(ptv0, gmv0, sgv0, sem0)

  return _topk_gather


_topk_half = _make_topk(B // 2)


def _leaky(h):
    return jnp.where(h >= 0, h, 0.2 * h)


def _scales_kernel(xg_ref, x_ref, acat_ref, dcat_ref, bcat_ref,
                   wa1_ref, ba1_ref, wa2_ref, ba2_ref,
                   fused_ref, w_ref):
    x = x_ref[0]
    acat = acat_ref[...]
    zcat = jnp.dot(dcat_ref[...], x, preferred_element_type=jnp.float32) \
        + bcat_ref[...]

    feats = []
    m = None
    for kk in range(KMAX):
        y = jnp.dot(acat, xg_ref[0, :, kk, :],
                    preferred_element_type=jnp.float32)
        m = y if m is None else jnp.maximum(m, y)
        if kk == 9:
            feats.append(_leaky(m[0:O] + zcat[0:O]))
        elif kk == 19:
            feats.append(_leaky(m[O:2 * O] + zcat[O:2 * O]))
        elif kk == 39:
            feats.append(_leaky(m[2 * O:3 * O] + zcat[2 * O:3 * O]))

    cc = jnp.concatenate(feats, axis=0)
    a = jnp.dot(wa1_ref[...], cc, preferred_element_type=jnp.float32) + ba1_ref[...]
    a = jnp.maximum(a, 0.0)
    logits = jnp.dot(wa2_ref[...], a, preferred_element_type=jnp.float32) + ba2_ref[...]
    lmax = jnp.max(logits, axis=0, keepdims=True)
    ex = jnp.exp(logits - lmax)
    w = ex / jnp.sum(ex, axis=0, keepdims=True)
    w_ref[0] = w
    fused = w[0:1] * feats[0] + w[1:2] * feats[1] + w[2:3] * feats[2]
    fused_ref[0] = fused


def _scales(xg, x, acat, dcat, bcat, wa1p, ba1p, wa2, ba2, hb):
    return pl.pallas_call(
        _scales_kernel,
        grid=(hb, N // T_C),
        in_specs=[
            pl.BlockSpec((1, C, KMAX, T_C), lambda b, t: (b, 0, 0, t)),
            pl.BlockSpec((1, C, T_C), lambda b, t: (b, 0, t)),
            pl.BlockSpec((3 * O, C), lambda b, t: (0, 0)),
            pl.BlockSpec((3 * O, C), lambda b, t: (0, 0)),
            pl.BlockSpec((3 * O, 1), lambda b, t: (0, 0)),
            pl.BlockSpec((O, 3 * O), lambda b, t: (0, 0)),
            pl.BlockSpec((O, 1), lambda b, t: (0, 0)),
            pl.BlockSpec((3, O), lambda b, t: (0, 0)),
            pl.BlockSpec((3, 1), lambda b, t: (0, 0)),
        ],
        out_specs=[
            pl.BlockSpec((1, O, T_C), lambda b, t: (b, 0, t)),
            pl.BlockSpec((1, 3, T_C), lambda b, t: (b, 0, t)),
        ],
        out_shape=[
            jax.ShapeDtypeStruct((hb, O, N), jnp.float32),
            jax.ShapeDtypeStruct((hb, 3, N), jnp.float32),
        ],
    )(xg, x, acat, dcat, bcat, wa1p, ba1p, wa2, ba2)


def kernel(x, W0, g0, b0, W1, g1, b1, W2, g2, b2, Wa1, ba1, ga, bba, Wa2, ba2):
    acat = jnp.concatenate([g0[:, None] * W0[:, :C],
                            g1[:, None] * W1[:, :C],
                            g2[:, None] * W2[:, :C]], axis=0)
    dcat = jnp.concatenate([g0[:, None] * (W0[:, C:] - W0[:, :C]),
                            g1[:, None] * (W1[:, C:] - W1[:, :C]),
                            g2[:, None] * (W2[:, C:] - W2[:, :C])], axis=0)
    bcat = jnp.concatenate([b0, b1, b2], axis=0)[:, None]
    wa1p = ga[:, None] * Wa1
    ba1p = (ga * ba1 + bba)[:, None]
    ba2c = ba2[:, None]

    xt = jnp.transpose(x, (0, 2, 1))
    hb = B // 2
    outs = []
    for h in range(2):
        xh = x[h * hb:(h + 1) * hb]
        xth = xt[h * hb:(h + 1) * hb]
        pt, gm, sg = _pairwise(xth, xh, hb)
        xg = _topk_half(pt, gm, sg, xh)
        outs.append(_scales(xg, xh, acat, dcat, bcat, wa1p, ba1p, Wa2, ba2c,
                            hb))
    fused = jnp.concatenate([outs[0][0], outs[1][0]], axis=0)
    w = jnp.concatenate([outs[0][1], outs[1][1]], axis=0)
    return fused, w
